# Initial kernel scaffold; baseline (speedup 1.0000x reference)
import jax
import jax.numpy as jnp
from jax.experimental import pallas as pl


def kernel(x, stem_w, stem_scale, stem_bias, g1_b0_pro_scale, g1_b0_pro_bias, g1_b0_conv0_w, g1_b0_conv0_scale, g1_b0_conv0_bias, g1_b0_conv1_w, g1_b0_conv1_scale, g1_b0_conv1_bias, g1_b0_sc_w, g1_b0_sc_scale, g1_b0_sc_bias, g1_b1_pro_scale, g1_b1_pro_bias, g1_b1_conv0_w, g1_b1_conv0_scale, g1_b1_conv0_bias, g1_b1_conv1_w, g1_b1_conv1_scale, g1_b1_conv1_bias, g2_b0_pro_scale, g2_b0_pro_bias, g2_b0_conv0_w, g2_b0_conv0_scale, g2_b0_conv0_bias, g2_b0_conv1_w, g2_b0_conv1_scale, g2_b0_conv1_bias, g2_b0_sc_w, g2_b0_sc_scale, g2_b0_sc_bias, g2_b1_pro_scale, g2_b1_pro_bias, g2_b1_conv0_w, g2_b1_conv0_scale, g2_b1_conv0_bias, g2_b1_conv1_w, g2_b1_conv1_scale, g2_b1_conv1_bias, g3_b0_pro_scale, g3_b0_pro_bias, g3_b0_conv0_w, g3_b0_conv0_scale, g3_b0_conv0_bias, g3_b0_conv1_w, g3_b0_conv1_scale, g3_b0_conv1_bias, g3_b0_sc_w, g3_b0_sc_scale, g3_b0_sc_bias, g3_b1_pro_scale, g3_b1_pro_bias, g3_b1_conv0_w, g3_b1_conv0_scale, g3_b1_conv0_bias, g3_b1_conv1_w, g3_b1_conv1_scale, g3_b1_conv1_bias, bn_scale, bn_bias, rbc_w):
    raise NotImplementedError("write your pallas kernel here")


# speedup vs baseline: 2.4938x; 2.4938x over previous
import jax
import jax.numpy as jnp
from jax.experimental import pallas as pl
from jax.experimental.pallas import tpu as pltpu

NB = 2

OFF128 = {"stem": 0, "g1b0c0": 72, "g1b0c1": 216, "g1b1c0": 1368,
          "g1b1c1": 2520, "g1b0sc": 3672}
OFF256 = {"g2b0c0": 0, "g2b0c1": 1152, "g2b1c0": 3456, "g2b1c1": 5760,
          "g2b0sc": 8064}
OFF512 = {"g3b0c0": 0, "g3b0c1": 2304, "g3b1c0": 6912, "g3b1c1": 11520,
          "g3b0sc": 16128}

VROW = {"stem": 0, "g1b0": 2, "g1b1": 10, "g2b0": 16, "g2b1": 24,
        "g3b0": 30, "g3b1": 38, "head": 44}


def _net_kernel(x_ref, w128_ref, w256_ref, w512_ref, vec_ref, rbc_ref, o_ref):
    f32 = jnp.float32
    bf16 = jnp.bfloat16

    def vrow(k, c):
        return vec_ref[k, :c]

    def conv3x3(xb, wref, off, srow, relu, res=None, stride=1):
        nb, h, w, c = xb.shape
        cout = wref.shape[-1]
        if stride == 1:
            ho, wo = h, w
            xp = jnp.pad(xb, ((0, 0), (1, 1), (0, 0), (0, 0)))
            zc = jnp.zeros((nb, h + 2, 1, c), xb.dtype)
            xm = jnp.concatenate([zc, xp[:, :, :-1, :]], axis=2)
            xq = jnp.concatenate([xp[:, :, 1:, :], zc], axis=2)
            x3 = jnp.concatenate([xm, xp, xq], axis=-1)

            def dygrp(dy):
                return x3[:, dy:dy + ho, :, :]
        else:
            ho, wo = h // 2, w // 2
            hh = (h + 2) // 2
            xr = xb.reshape(nb, h, wo, 2, c)
            e0 = xr[:, :, :, 0, :]
            o1 = xr[:, :, :, 1, :]
            zc = jnp.zeros((nb, h, 1, c), xb.dtype)
            om = jnp.concatenate([zc, o1[:, :, :-1, :]], axis=2)
            x3 = jnp.concatenate([om, e0, o1], axis=-1)
            x3 = jnp.pad(x3, ((0, 0), (1, 1), (0, 0), (0, 0)))
            x3 = x3.reshape(nb, hh, 2, wo, 3 * c)

            def dygrp(dy):
                return x3[:, dy // 2:dy // 2 + ho, dy % 2, :, :]

        m = nb * ho * wo
        k3 = 3 * c
        acc = jnp.zeros((m, cout), f32)
        for dy in range(3):
            wt = wref[off + dy * k3: off + (dy + 1) * k3, :]
            acc = acc + jnp.dot(dygrp(dy).reshape(m, k3), wt,
                                preferred_element_type=f32)
        y = acc * vrow(srow, cout).reshape(1, cout) \
            + vrow(srow + 1, cout).reshape(1, cout)
        if relu:
            y = jnp.maximum(y, 0.0)
        if res is not None:
            y = y + res.reshape(m, cout).astype(f32)
        return y.reshape(nb, ho, wo, cout).astype(bf16)

    def conv1x1(xb, wref, off, srow):
        nb, h, w, c = xb.shape
        cout = wref.shape[-1]
        m = nb * h * w
        acc = jnp.dot(xb.reshape(m, c), wref[off:off + c, :],
                      preferred_element_type=f32)
        y = acc * vrow(srow, cout).reshape(1, cout) \
            + vrow(srow + 1, cout).reshape(1, cout)
        return y.reshape(nb, h, w, cout).astype(bf16)

    def resblock(h_in, wref, c0_off, c1_off, sc_off, vk, stride):
        nb, h, w, c = h_in.shape
        z = h_in.astype(f32) * vrow(vk, c).reshape(1, 1, 1, c) \
            + vrow(vk + 1, c).reshape(1, 1, 1, c)
        z = jnp.maximum(z, 0.0).astype(bf16)
        c0 = conv3x3(z, wref, c0_off, vk + 2, relu=True, stride=stride)
        if sc_off is not None:
            if stride == 1:
                zs = z
            else:
                zs = z.reshape(nb, h // 2, 2, w, c)[:, :, 0, :, :]
                zs = zs.reshape(nb, h // 2, w // 2, 2, c)[:, :, :, 0, :]
            res = conv1x1(zs, wref, sc_off, vk + 6)
        else:
            res = h_in
        return conv3x3(c0, wref, c1_off, vk + 4, relu=False, res=res)

    x72 = x_ref[...]
    nb = x72.shape[0]
    acc = jnp.dot(x72.reshape(nb * 32 * 32, 72), w128_ref[0:72, 0:16],
                  preferred_element_type=f32)
    h_act = acc * vrow(VROW["stem"], 16).reshape(1, 16) \
        + vrow(VROW["stem"] + 1, 16).reshape(1, 16)
    h_act = h_act.reshape(nb, 32, 32, 16).astype(bf16)

    h_act = resblock(h_act, w128_ref, OFF128["g1b0c0"], OFF128["g1b0c1"],
                     OFF128["g1b0sc"], VROW["g1b0"], stride=1)
    h_act = resblock(h_act, w128_ref, OFF128["g1b1c0"], OFF128["g1b1c1"],
                     None, VROW["g1b1"], stride=1)
    h_act = resblock(h_act, w256_ref, OFF256["g2b0c0"], OFF256["g2b0c1"],
                     OFF256["g2b0sc"], VROW["g2b0"], stride=2)
    h_act = resblock(h_act, w256_ref, OFF256["g2b1c0"], OFF256["g2b1c1"],
                     None, VROW["g2b1"], stride=1)
    h_act = resblock(h_act, w512_ref, OFF512["g3b0c0"], OFF512["g3b0c1"],
                     OFF512["g3b0sc"], VROW["g3b0"], stride=2)
    h_act = resblock(h_act, w512_ref, OFF512["g3b1c0"], OFF512["g3b1c1"],
                     None, VROW["g3b1"], stride=1)

    hf = h_act.astype(f32).reshape(nb, 64, 512)
    pooled = jnp.sum(hf, axis=1) * (1.0 / 64.0)
    feat = jnp.tanh(pooled * vrow(VROW["head"], 512).reshape(1, 512)
                    + vrow(VROW["head"] + 1, 512).reshape(1, 512))
    o_ref[...] = jnp.dot(feat, rbc_ref[...],
                         preferred_element_type=f32).reshape(1, nb, 128)


def _pad512(v):
    return jnp.pad(v.astype(jnp.float32), (0, 512 - v.shape[0]))


def kernel(x, stem_w, stem_scale, stem_bias, g1_b0_pro_scale, g1_b0_pro_bias, g1_b0_conv0_w, g1_b0_conv0_scale, g1_b0_conv0_bias, g1_b0_conv1_w, g1_b0_conv1_scale, g1_b0_conv1_bias, g1_b0_sc_w, g1_b0_sc_scale, g1_b0_sc_bias, g1_b1_pro_scale, g1_b1_pro_bias, g1_b1_conv0_w, g1_b1_conv0_scale, g1_b1_conv0_bias, g1_b1_conv1_w, g1_b1_conv1_scale, g1_b1_conv1_bias, g2_b0_pro_scale, g2_b0_pro_bias, g2_b0_conv0_w, g2_b0_conv0_scale, g2_b0_conv0_bias, g2_b0_conv1_w, g2_b0_conv1_scale, g2_b0_conv1_bias, g2_b0_sc_w, g2_b0_sc_scale, g2_b0_sc_bias, g2_b1_pro_scale, g2_b1_pro_bias, g2_b1_conv0_w, g2_b1_conv0_scale, g2_b1_conv0_bias, g2_b1_conv1_w, g2_b1_conv1_scale, g2_b1_conv1_bias, g3_b0_pro_scale, g3_b0_pro_bias, g3_b0_conv0_w, g3_b0_conv0_scale, g3_b0_conv0_bias, g3_b0_conv1_w, g3_b0_conv1_scale, g3_b0_conv1_bias, g3_b0_sc_w, g3_b0_sc_scale, g3_b0_sc_bias, g3_b1_pro_scale, g3_b1_pro_bias, g3_b1_conv0_w, g3_b1_conv0_scale, g3_b1_conv0_bias, g3_b1_conv1_w, g3_b1_conv1_scale, g3_b1_conv1_bias, bn_scale, bn_bias, rbc_w):
    n = x.shape[0]

    xh = jnp.transpose(x, (0, 2, 3, 1)).astype(jnp.bfloat16)
    xh = jnp.pad(xh, ((0, 0), (1, 1), (1, 1), (0, 5)))
    xh = jnp.concatenate(
        [xh[:, dy:dy + 32, dx:dx + 32, :]
         for dy in range(3) for dx in range(3)], axis=-1)

    w128 = jnp.concatenate([
        stem_w[:, :8, :].reshape(72, 128),
        g1_b0_conv0_w[:, :16, :].reshape(144, 128),
        g1_b0_conv1_w.reshape(1152, 128),
        g1_b1_conv0_w.reshape(1152, 128),
        g1_b1_conv1_w.reshape(1152, 128),
        g1_b0_sc_w[:, :16, :].reshape(16, 128),
    ], axis=0)
    w256 = jnp.concatenate([
        g2_b0_conv0_w.reshape(1152, 256),
        g2_b0_conv1_w.reshape(2304, 256),
        g2_b1_conv0_w.reshape(2304, 256),
        g2_b1_conv1_w.reshape(2304, 256),
        g2_b0_sc_w.reshape(128, 256),
    ], axis=0)
    w512 = jnp.concatenate([
        g3_b0_conv0_w.reshape(2304, 512),
        g3_b0_conv1_w.reshape(4608, 512),
        g3_b1_conv0_w.reshape(4608, 512),
        g3_b1_conv1_w.reshape(4608, 512),
        g3_b0_sc_w.reshape(256, 512),
    ], axis=0)

    rows = [
        stem_scale, stem_bias,
        g1_b0_pro_scale, g1_b0_pro_bias, g1_b0_conv0_scale, g1_b0_conv0_bias,
        g1_b0_conv1_scale, g1_b0_conv1_bias, g1_b0_sc_scale, g1_b0_sc_bias,
        g1_b1_pro_scale, g1_b1_pro_bias, g1_b1_conv0_scale, g1_b1_conv0_bias,
        g1_b1_conv1_scale, g1_b1_conv1_bias,
        g2_b0_pro_scale, g2_b0_pro_bias, g2_b0_conv0_scale, g2_b0_conv0_bias,
        g2_b0_conv1_scale, g2_b0_conv1_bias, g2_b0_sc_scale, g2_b0_sc_bias,
        g2_b1_pro_scale, g2_b1_pro_bias, g2_b1_conv0_scale, g2_b1_conv0_bias,
        g2_b1_conv1_scale, g2_b1_conv1_bias,
        g3_b0_pro_scale, g3_b0_pro_bias, g3_b0_conv0_scale, g3_b0_conv0_bias,
        g3_b0_conv1_scale, g3_b0_conv1_bias, g3_b0_sc_scale, g3_b0_sc_bias,
        g3_b1_pro_scale, g3_b1_pro_bias, g3_b1_conv0_scale, g3_b1_conv0_bias,
        g3_b1_conv1_scale, g3_b1_conv1_bias,
        bn_scale, bn_bias,
    ]
    vecs = jnp.stack([_pad512(r) for r in rows]
                     + [jnp.zeros((512,), jnp.float32)] * 2)

    out = pl.pallas_call(
        _net_kernel,
        grid=(n // NB,),
        in_specs=[
            pl.BlockSpec((NB, 32, 32, 72), lambda i: (i, 0, 0, 0)),
            pl.BlockSpec(w128.shape, lambda i: (0, 0)),
            pl.BlockSpec(w256.shape, lambda i: (0, 0)),
            pl.BlockSpec(w512.shape, lambda i: (0, 0)),
            pl.BlockSpec((48, 512), lambda i: (0, 0)),
            pl.BlockSpec((512, 128), lambda i: (0, 0)),
        ],
        out_specs=pl.BlockSpec((1, NB, 128), lambda i: (i, 0, 0)),
        out_shape=jax.ShapeDtypeStruct((n // NB, NB, 128), jnp.float32),
        compiler_params=pltpu.CompilerParams(
            dimension_semantics=("parallel",),
            vmem_limit_bytes=64 * 1024 * 1024),
    )(xh, w128, w256, w512, vecs, rbc_w)
    return out.reshape(n, 128)[:, :10]


# speedup vs baseline: 3.3977x; 1.3625x over previous
import jax
import jax.numpy as jnp
from jax.experimental import pallas as pl
from jax.experimental.pallas import tpu as pltpu

NB = 2

OFF128 = {"stem": 0, "g1b0c0": 32, "g1b0c1": 176, "g1b1c0": 1328,
          "g1b1c1": 2480, "g1b0sc": 3632}
OFF256 = {"g2b0c0": 0, "g2b0c1": 1152, "g2b1c0": 3456, "g2b1c1": 5760,
          "g2b0sc": 8064}
OFF512 = {"g3b0c0": 0, "g3b0c1": 2304, "g3b1c0": 6912, "g3b1c1": 11520,
          "g3b0sc": 16128}

VROW = {"stem": 0, "g1b0": 2, "g1b1": 10, "g2b0": 16, "g2b1": 24,
        "g3b0": 30, "g3b1": 38, "head": 44}


def _net_kernel(x_ref, w128_ref, w256_ref, w512_ref, vec_ref, rbc_ref, o_ref):
    f32 = jnp.float32
    bf16 = jnp.bfloat16

    def vrow(k, c):
        return vec_ref[k, :c]

    def conv3x3(xb, wref, off, srow, relu, res=None, stride=1):
        nb, h, w, c = xb.shape
        cout = wref.shape[-1]
        if stride == 1:
            ho, wo = h, w
            xp = jnp.pad(xb, ((0, 0), (1, 1), (0, 0), (0, 0)))
            zc = jnp.zeros((nb, h + 2, 1, c), xb.dtype)
            xm = jnp.concatenate([zc, xp[:, :, :-1, :]], axis=2)
            xq = jnp.concatenate([xp[:, :, 1:, :], zc], axis=2)
            x3 = jnp.concatenate([xm, xp, xq], axis=-1)

            def dygrp(dy):
                return x3[:, dy:dy + ho, :, :]
        else:
            ho, wo = h // 2, w // 2
            hh = (h + 2) // 2
            xr = xb.reshape(nb, h, wo, 2, c)
            e0 = xr[:, :, :, 0, :]
            o1 = xr[:, :, :, 1, :]
            zc = jnp.zeros((nb, h, 1, c), xb.dtype)
            om = jnp.concatenate([zc, o1[:, :, :-1, :]], axis=2)
            x3 = jnp.concatenate([om, e0, o1], axis=-1)
            x3 = jnp.pad(x3, ((0, 0), (1, 1), (0, 0), (0, 0)))
            x3 = x3.reshape(nb, hh, 2, wo, 3 * c)

            def dygrp(dy):
                return x3[:, dy // 2:dy // 2 + ho, dy % 2, :, :]

        m = nb * ho * wo
        k3 = 3 * c
        acc = jnp.zeros((m, cout), f32)
        for dy in range(3):
            wt = wref[off + dy * k3: off + (dy + 1) * k3, :]
            acc = acc + jnp.dot(dygrp(dy).reshape(m, k3), wt,
                                preferred_element_type=f32)
        y = acc * vrow(srow, cout).reshape(1, cout) \
            + vrow(srow + 1, cout).reshape(1, cout)
        if relu:
            y = jnp.maximum(y, 0.0)
        if res is not None:
            y = y + res.reshape(m, cout).astype(f32)
        return y.reshape(nb, ho, wo, cout).astype(bf16)

    def conv1x1(xb, wref, off, srow):
        nb, h, w, c = xb.shape
        cout = wref.shape[-1]
        m = nb * h * w
        acc = jnp.dot(xb.reshape(m, c), wref[off:off + c, :],
                      preferred_element_type=f32)
        y = acc * vrow(srow, cout).reshape(1, cout) \
            + vrow(srow + 1, cout).reshape(1, cout)
        return y.reshape(nb, h, w, cout).astype(bf16)

    def resblock(h_in, wref, c0_off, c1_off, sc_off, vk, stride):
        nb, h, w, c = h_in.shape
        z = h_in.astype(f32) * vrow(vk, c).reshape(1, 1, 1, c) \
            + vrow(vk + 1, c).reshape(1, 1, 1, c)
        z = jnp.maximum(z, 0.0).astype(bf16)
        c0 = conv3x3(z, wref, c0_off, vk + 2, relu=True, stride=stride)
        if sc_off is not None:
            if stride == 1:
                zs = z
            else:
                zs = z.reshape(nb, h // 2, 2, w, c)[:, :, 0, :, :]
                zs = zs.reshape(nb, h // 2, w // 2, 2, c)[:, :, :, 0, :]
            res = conv1x1(zs, wref, sc_off, vk + 6)
        else:
            res = h_in
        return conv3x3(c0, wref, c1_off, vk + 4, relu=False, res=res)

    xr = x_ref[...]
    nb = xr.shape[0]
    xt = jnp.transpose(xr, (0, 2, 3, 1)).astype(bf16)
    xtp = jnp.pad(xt, ((0, 0), (1, 1), (1, 1), (0, 0)))
    cols = jnp.concatenate(
        [xtp[:, dy:dy + 32, dx:dx + 32, :]
         for dy in range(3) for dx in range(3)]
        + [jnp.zeros((nb, 32, 32, 5), bf16)], axis=-1)
    acc = jnp.dot(cols.reshape(nb * 32 * 32, 32), w128_ref[0:32, 0:16],
                  preferred_element_type=f32)
    h_act = acc * vrow(VROW["stem"], 16).reshape(1, 16) \
        + vrow(VROW["stem"] + 1, 16).reshape(1, 16)
    h_act = h_act.reshape(nb, 32, 32, 16).astype(bf16)

    h_act = resblock(h_act, w128_ref, OFF128["g1b0c0"], OFF128["g1b0c1"],
                     OFF128["g1b0sc"], VROW["g1b0"], stride=1)
    h_act = resblock(h_act, w128_ref, OFF128["g1b1c0"], OFF128["g1b1c1"],
                     None, VROW["g1b1"], stride=1)
    h_act = resblock(h_act, w256_ref, OFF256["g2b0c0"], OFF256["g2b0c1"],
                     OFF256["g2b0sc"], VROW["g2b0"], stride=2)
    h_act = resblock(h_act, w256_ref, OFF256["g2b1c0"], OFF256["g2b1c1"],
                     None, VROW["g2b1"], stride=1)
    h_act = resblock(h_act, w512_ref, OFF512["g3b0c0"], OFF512["g3b0c1"],
                     OFF512["g3b0sc"], VROW["g3b0"], stride=2)
    h_act = resblock(h_act, w512_ref, OFF512["g3b1c0"], OFF512["g3b1c1"],
                     None, VROW["g3b1"], stride=1)

    hf = h_act.astype(f32).reshape(nb, 64, 512)
    pooled = jnp.sum(hf, axis=1) * (1.0 / 64.0)
    feat = jnp.tanh(pooled * vrow(VROW["head"], 512).reshape(1, 512)
                    + vrow(VROW["head"] + 1, 512).reshape(1, 512))
    o_ref[...] = jnp.dot(feat, rbc_ref[...],
                         preferred_element_type=f32).reshape(1, nb, 128)


def _pad512(v):
    return jnp.pad(v.astype(jnp.float32), (0, 512 - v.shape[0]))


def kernel(x, stem_w, stem_scale, stem_bias, g1_b0_pro_scale, g1_b0_pro_bias, g1_b0_conv0_w, g1_b0_conv0_scale, g1_b0_conv0_bias, g1_b0_conv1_w, g1_b0_conv1_scale, g1_b0_conv1_bias, g1_b0_sc_w, g1_b0_sc_scale, g1_b0_sc_bias, g1_b1_pro_scale, g1_b1_pro_bias, g1_b1_conv0_w, g1_b1_conv0_scale, g1_b1_conv0_bias, g1_b1_conv1_w, g1_b1_conv1_scale, g1_b1_conv1_bias, g2_b0_pro_scale, g2_b0_pro_bias, g2_b0_conv0_w, g2_b0_conv0_scale, g2_b0_conv0_bias, g2_b0_conv1_w, g2_b0_conv1_scale, g2_b0_conv1_bias, g2_b0_sc_w, g2_b0_sc_scale, g2_b0_sc_bias, g2_b1_pro_scale, g2_b1_pro_bias, g2_b1_conv0_w, g2_b1_conv0_scale, g2_b1_conv0_bias, g2_b1_conv1_w, g2_b1_conv1_scale, g2_b1_conv1_bias, g3_b0_pro_scale, g3_b0_pro_bias, g3_b0_conv0_w, g3_b0_conv0_scale, g3_b0_conv0_bias, g3_b0_conv1_w, g3_b0_conv1_scale, g3_b0_conv1_bias, g3_b0_sc_w, g3_b0_sc_scale, g3_b0_sc_bias, g3_b1_pro_scale, g3_b1_pro_bias, g3_b1_conv0_w, g3_b1_conv0_scale, g3_b1_conv0_bias, g3_b1_conv1_w, g3_b1_conv1_scale, g3_b1_conv1_bias, bn_scale, bn_bias, rbc_w):
    n = x.shape[0]

    w128 = jnp.concatenate([
        jnp.pad(stem_w[:, :3, :].reshape(27, 128), ((0, 5), (0, 0))),
        g1_b0_conv0_w[:, :16, :].reshape(144, 128),
        g1_b0_conv1_w.reshape(1152, 128),
        g1_b1_conv0_w.reshape(1152, 128),
        g1_b1_conv1_w.reshape(1152, 128),
        g1_b0_sc_w[:, :16, :].reshape(16, 128),
    ], axis=0)
    w256 = jnp.concatenate([
        g2_b0_conv0_w.reshape(1152, 256),
        g2_b0_conv1_w.reshape(2304, 256),
        g2_b1_conv0_w.reshape(2304, 256),
        g2_b1_conv1_w.reshape(2304, 256),
        g2_b0_sc_w.reshape(128, 256),
    ], axis=0)
    w512 = jnp.concatenate([
        g3_b0_conv0_w.reshape(2304, 512),
        g3_b0_conv1_w.reshape(4608, 512),
        g3_b1_conv0_w.reshape(4608, 512),
        g3_b1_conv1_w.reshape(4608, 512),
        g3_b0_sc_w.reshape(256, 512),
    ], axis=0)

    rows = [
        stem_scale, stem_bias,
        g1_b0_pro_scale, g1_b0_pro_bias, g1_b0_conv0_scale, g1_b0_conv0_bias,
        g1_b0_conv1_scale, g1_b0_conv1_bias, g1_b0_sc_scale, g1_b0_sc_bias,
        g1_b1_pro_scale, g1_b1_pro_bias, g1_b1_conv0_scale, g1_b1_conv0_bias,
        g1_b1_conv1_scale, g1_b1_conv1_bias,
        g2_b0_pro_scale, g2_b0_pro_bias, g2_b0_conv0_scale, g2_b0_conv0_bias,
        g2_b0_conv1_scale, g2_b0_conv1_bias, g2_b0_sc_scale, g2_b0_sc_bias,
        g2_b1_pro_scale, g2_b1_pro_bias, g2_b1_conv0_scale, g2_b1_conv0_bias,
        g2_b1_conv1_scale, g2_b1_conv1_bias,
        g3_b0_pro_scale, g3_b0_pro_bias, g3_b0_conv0_scale, g3_b0_conv0_bias,
        g3_b0_conv1_scale, g3_b0_conv1_bias, g3_b0_sc_scale, g3_b0_sc_bias,
        g3_b1_pro_scale, g3_b1_pro_bias, g3_b1_conv0_scale, g3_b1_conv0_bias,
        g3_b1_conv1_scale, g3_b1_conv1_bias,
        bn_scale, bn_bias,
    ]
    vecs = jnp.stack([_pad512(r) for r in rows]
                     + [jnp.zeros((512,), jnp.float32)] * 2)

    out = pl.pallas_call(
        _net_kernel,
        grid=(n // NB,),
        in_specs=[
            pl.BlockSpec((NB, 3, 32, 32), lambda i: (i, 0, 0, 0)),
            pl.BlockSpec(w128.shape, lambda i: (0, 0)),
            pl.BlockSpec(w256.shape, lambda i: (0, 0)),
            pl.BlockSpec(w512.shape, lambda i: (0, 0)),
            pl.BlockSpec((48, 512), lambda i: (0, 0)),
            pl.BlockSpec((512, 128), lambda i: (0, 0)),
        ],
        out_specs=pl.BlockSpec((1, NB, 128), lambda i: (i, 0, 0)),
        out_shape=jax.ShapeDtypeStruct((n // NB, NB, 128), jnp.float32),
        compiler_params=pltpu.CompilerParams(
            dimension_semantics=("parallel",),
            vmem_limit_bytes=64 * 1024 * 1024),
    )(x, w128, w256, w512, vecs, rbc_w)
    return out.reshape(n, 128)[:, :10]


# speedup vs baseline: 3.4398x; 1.0124x over previous
import jax
import jax.numpy as jnp
from jax.experimental import pallas as pl
from jax.experimental.pallas import tpu as pltpu

NB = 2

OFF128 = {"stem": 0, "g1b0c0": 32, "g1b0c1": 176, "g1b1c0": 1328,
          "g1b1c1": 2480, "g1b0sc": 3632}
OFF256 = {"g2b0c0": 0, "g2b0c1": 1152, "g2b1c0": 3456, "g2b1c1": 5760,
          "g2b0sc": 8064}
OFF512 = {"g3b0c0": 0, "g3b0c1": 2304, "g3b1c0": 6912, "g3b1c1": 11520,
          "g3b0sc": 16128}

VROW = {"stem": 0, "g1b0": 2, "g1b1": 10, "g2b0": 16, "g2b1": 24,
        "g3b0": 30, "g3b1": 38, "head": 44}


def _net_kernel(x_ref, w128_ref, w256_ref, w512_ref, vec_ref, rbc_ref, o_ref):
    f32 = jnp.float32
    bf16 = jnp.bfloat16

    def vrow(k, c):
        return vec_ref[k, :c]

    def conv3x3(xb, wref, off, srow, relu, res=None, stride=1):
        nb, h, w, c = xb.shape
        cout = wref.shape[-1]
        if stride == 1:
            ho, wo = h, w
            xp = jnp.pad(xb, ((0, 0), (1, 1), (0, 0), (0, 0)))
            zc = jnp.zeros((nb, h + 2, 1, c), xb.dtype)
            xm = jnp.concatenate([zc, xp[:, :, :-1, :]], axis=2)
            xq = jnp.concatenate([xp[:, :, 1:, :], zc], axis=2)
            x3 = jnp.concatenate([xm, xp, xq], axis=-1)

            def dygrp(dy):
                return x3[:, dy:dy + ho, :, :]
        else:
            ho, wo = h // 2, w // 2
            hh = (h + 2) // 2
            xr = xb.reshape(nb, h, wo, 2, c)
            e0 = xr[:, :, :, 0, :]
            o1 = xr[:, :, :, 1, :]
            zc = jnp.zeros((nb, h, 1, c), xb.dtype)
            om = jnp.concatenate([zc, o1[:, :, :-1, :]], axis=2)
            x3 = jnp.concatenate([om, e0, o1], axis=-1)
            x3 = jnp.pad(x3, ((0, 0), (1, 1), (0, 0), (0, 0)))
            x3 = x3.reshape(nb, hh, 2, wo, 3 * c)

            def dygrp(dy):
                return x3[:, dy // 2:dy // 2 + ho, dy % 2, :, :]

        m = nb * ho * wo
        k3 = 3 * c
        acc = jnp.zeros((m, cout), f32)
        for dy in range(3):
            wt = wref[off + dy * k3: off + (dy + 1) * k3, :]
            acc = acc + jnp.dot(dygrp(dy).reshape(m, k3), wt,
                                preferred_element_type=f32)
        y = acc * vrow(srow, cout).reshape(1, cout) \
            + vrow(srow + 1, cout).reshape(1, cout)
        if relu:
            y = jnp.maximum(y, 0.0)
        if res is not None:
            y = y + res.reshape(m, cout).astype(f32)
        return y.reshape(nb, ho, wo, cout).astype(bf16)

    def conv1x1(xb, wref, off, srow):
        nb, h, w, c = xb.shape
        cout = wref.shape[-1]
        m = nb * h * w
        acc = jnp.dot(xb.reshape(m, c), wref[off:off + c, :],
                      preferred_element_type=f32)
        y = acc * vrow(srow, cout).reshape(1, cout) \
            + vrow(srow + 1, cout).reshape(1, cout)
        return y.reshape(nb, h, w, cout).astype(bf16)

    def resblock(h_in, wref, c0_off, c1_off, sc_off, vk, stride):
        nb, h, w, c = h_in.shape
        z = h_in.astype(f32) * vrow(vk, c).reshape(1, 1, 1, c) \
            + vrow(vk + 1, c).reshape(1, 1, 1, c)
        z = jnp.maximum(z, 0.0).astype(bf16)
        c0 = conv3x3(z, wref, c0_off, vk + 2, relu=True, stride=stride)
        if sc_off is not None:
            if stride == 1:
                zs = z
            else:
                zs = z.reshape(nb, h // 2, 2, w, c)[:, :, 0, :, :]
                zs = zs.reshape(nb, h // 2, w // 2, 2, c)[:, :, :, 0, :]
            res = conv1x1(zs, wref, sc_off, vk + 6)
        else:
            res = h_in
        return conv3x3(c0, wref, c1_off, vk + 4, relu=False, res=res)

    xr = x_ref[...]
    nb = xr.shape[0]
    xt = jnp.transpose(xr.astype(bf16), (0, 2, 3, 1))
    xtp = jnp.pad(xt, ((0, 0), (1, 1), (1, 1), (0, 0)))
    cols = jnp.concatenate(
        [xtp[:, dy:dy + 32, dx:dx + 32, :]
         for dy in range(3) for dx in range(3)]
        + [jnp.zeros((nb, 32, 32, 5), bf16)], axis=-1)
    acc = jnp.dot(cols.reshape(nb * 32 * 32, 32), w128_ref[0:32, 0:16],
                  preferred_element_type=f32)
    h_act = acc * vrow(VROW["stem"], 16).reshape(1, 16) \
        + vrow(VROW["stem"] + 1, 16).reshape(1, 16)
    h_act = h_act.reshape(nb, 32, 32, 16).astype(bf16)

    h_act = resblock(h_act, w128_ref, OFF128["g1b0c0"], OFF128["g1b0c1"],
                     OFF128["g1b0sc"], VROW["g1b0"], stride=1)
    h_act = resblock(h_act, w128_ref, OFF128["g1b1c0"], OFF128["g1b1c1"],
                     None, VROW["g1b1"], stride=1)
    h_act = resblock(h_act, w256_ref, OFF256["g2b0c0"], OFF256["g2b0c1"],
                     OFF256["g2b0sc"], VROW["g2b0"], stride=2)
    h_act = resblock(h_act, w256_ref, OFF256["g2b1c0"], OFF256["g2b1c1"],
                     None, VROW["g2b1"], stride=1)
    h_act = resblock(h_act, w512_ref, OFF512["g3b0c0"], OFF512["g3b0c1"],
                     OFF512["g3b0sc"], VROW["g3b0"], stride=2)
    h_act = resblock(h_act, w512_ref, OFF512["g3b1c0"], OFF512["g3b1c1"],
                     None, VROW["g3b1"], stride=1)

    hf = h_act.astype(f32).reshape(nb, 64, 512)
    pooled = jnp.sum(hf, axis=1) * (1.0 / 64.0)
    feat = jnp.tanh(pooled * vrow(VROW["head"], 512).reshape(1, 512)
                    + vrow(VROW["head"] + 1, 512).reshape(1, 512))
    o_ref[...] = jnp.dot(feat, rbc_ref[...],
                         preferred_element_type=f32).reshape(1, nb, 128)


def _pad512(v):
    return jnp.pad(v.astype(jnp.float32), (0, 512 - v.shape[0]))


def kernel(x, stem_w, stem_scale, stem_bias, g1_b0_pro_scale, g1_b0_pro_bias, g1_b0_conv0_w, g1_b0_conv0_scale, g1_b0_conv0_bias, g1_b0_conv1_w, g1_b0_conv1_scale, g1_b0_conv1_bias, g1_b0_sc_w, g1_b0_sc_scale, g1_b0_sc_bias, g1_b1_pro_scale, g1_b1_pro_bias, g1_b1_conv0_w, g1_b1_conv0_scale, g1_b1_conv0_bias, g1_b1_conv1_w, g1_b1_conv1_scale, g1_b1_conv1_bias, g2_b0_pro_scale, g2_b0_pro_bias, g2_b0_conv0_w, g2_b0_conv0_scale, g2_b0_conv0_bias, g2_b0_conv1_w, g2_b0_conv1_scale, g2_b0_conv1_bias, g2_b0_sc_w, g2_b0_sc_scale, g2_b0_sc_bias, g2_b1_pro_scale, g2_b1_pro_bias, g2_b1_conv0_w, g2_b1_conv0_scale, g2_b1_conv0_bias, g2_b1_conv1_w, g2_b1_conv1_scale, g2_b1_conv1_bias, g3_b0_pro_scale, g3_b0_pro_bias, g3_b0_conv0_w, g3_b0_conv0_scale, g3_b0_conv0_bias, g3_b0_conv1_w, g3_b0_conv1_scale, g3_b0_conv1_bias, g3_b0_sc_w, g3_b0_sc_scale, g3_b0_sc_bias, g3_b1_pro_scale, g3_b1_pro_bias, g3_b1_conv0_w, g3_b1_conv0_scale, g3_b1_conv0_bias, g3_b1_conv1_w, g3_b1_conv1_scale, g3_b1_conv1_bias, bn_scale, bn_bias, rbc_w):
    n = x.shape[0]

    w128 = jnp.concatenate([
        jnp.pad(stem_w[:, :3, :].reshape(27, 128), ((0, 5), (0, 0))),
        g1_b0_conv0_w[:, :16, :].reshape(144, 128),
        g1_b0_conv1_w.reshape(1152, 128),
        g1_b1_conv0_w.reshape(1152, 128),
        g1_b1_conv1_w.reshape(1152, 128),
        g1_b0_sc_w[:, :16, :].reshape(16, 128),
    ], axis=0)
    w256 = jnp.concatenate([
        g2_b0_conv0_w.reshape(1152, 256),
        g2_b0_conv1_w.reshape(2304, 256),
        g2_b1_conv0_w.reshape(2304, 256),
        g2_b1_conv1_w.reshape(2304, 256),
        g2_b0_sc_w.reshape(128, 256),
    ], axis=0)
    w512 = jnp.concatenate([
        g3_b0_conv0_w.reshape(2304, 512),
        g3_b0_conv1_w.reshape(4608, 512),
        g3_b1_conv0_w.reshape(4608, 512),
        g3_b1_conv1_w.reshape(4608, 512),
        g3_b0_sc_w.reshape(256, 512),
    ], axis=0)

    rows = [
        stem_scale, stem_bias,
        g1_b0_pro_scale, g1_b0_pro_bias, g1_b0_conv0_scale, g1_b0_conv0_bias,
        g1_b0_conv1_scale, g1_b0_conv1_bias, g1_b0_sc_scale, g1_b0_sc_bias,
        g1_b1_pro_scale, g1_b1_pro_bias, g1_b1_conv0_scale, g1_b1_conv0_bias,
        g1_b1_conv1_scale, g1_b1_conv1_bias,
        g2_b0_pro_scale, g2_b0_pro_bias, g2_b0_conv0_scale, g2_b0_conv0_bias,
        g2_b0_conv1_scale, g2_b0_conv1_bias, g2_b0_sc_scale, g2_b0_sc_bias,
        g2_b1_pro_scale, g2_b1_pro_bias, g2_b1_conv0_scale, g2_b1_conv0_bias,
        g2_b1_conv1_scale, g2_b1_conv1_bias,
        g3_b0_pro_scale, g3_b0_pro_bias, g3_b0_conv0_scale, g3_b0_conv0_bias,
        g3_b0_conv1_scale, g3_b0_conv1_bias, g3_b0_sc_scale, g3_b0_sc_bias,
        g3_b1_pro_scale, g3_b1_pro_bias, g3_b1_conv0_scale, g3_b1_conv0_bias,
        g3_b1_conv1_scale, g3_b1_conv1_bias,
        bn_scale, bn_bias,
    ]
    vecs = jnp.stack([_pad512(r) for r in rows]
                     + [jnp.zeros((512,), jnp.float32)] * 2)

    out = pl.pallas_call(
        _net_kernel,
        grid=(n // NB,),
        in_specs=[
            pl.BlockSpec((NB, 3, 32, 32), lambda i: (i, 0, 0, 0)),
            pl.BlockSpec(w128.shape, lambda i: (0, 0)),
            pl.BlockSpec(w256.shape, lambda i: (0, 0)),
            pl.BlockSpec(w512.shape, lambda i: (0, 0)),
            pl.BlockSpec((48, 512), lambda i: (0, 0)),
            pl.BlockSpec((512, 128), lambda i: (0, 0)),
        ],
        out_specs=pl.BlockSpec((1, NB, 128), lambda i: (i, 0, 0)),
        out_shape=jax.ShapeDtypeStruct((n // NB, NB, 128), jnp.float32),
        compiler_params=pltpu.CompilerParams(
            dimension_semantics=("parallel",),
            vmem_limit_bytes=64 * 1024 * 1024),
    )(x, w128, w256, w512, vecs, rbc_w)
    return out.reshape(n, 128)[:, :10]
